# baseline (speedup 1.0000x reference)
import functools
import jax
import jax.numpy as jnp
from jax import lax
from jax.experimental import pallas as pl
from jax.experimental.pallas import tpu as pltpu
from jax.experimental.pallas import tpu_sc as plsc

N_ATOMS = 10000
N_BONDS = 320000
MAX_NB = 32
ATOM_FDIM = 128
BOND_FDIM = 144
HIDDEN = 128
N_MOLS = 100
ATOMS_PER_MOL = 100

NC, NS, L = 2, 16, 16
NW = NC * NS

A_PAD = 10240
A_PER_W = A_PAD // NW
SUB = 4
AGG_ROWS = SUB * MAX_NB
AGG_ITERS = A_PER_W // SUB

B_PER_W = N_BONDS // NW
CH = 80
BOND_ITERS = B_PER_W // CH

HG = HIDDEN // L


def _sc_mesh():
  return plsc.VectorSubcoreMesh(
      core_axis_name="c", subcore_axis_name="s", num_cores=NC, num_subcores=NS)


def _wid():
  return lax.axis_index("s") * NC + lax.axis_index("c")


def _agg_body(z_hbm, a2b_hbm, agg_hbm, idx_v, rows_v, out_v, sem):
  base = _wid() * A_PER_W

  @pl.loop(0, AGG_ITERS)
  def _(j):
    a0 = base + j * SUB
    pltpu.sync_copy(a2b_hbm.at[pl.ds(a0 * MAX_NB, AGG_ROWS)], idx_v)
    pltpu.async_copy(z_hbm.at[idx_v], rows_v, sem).wait()
    for i in range(SUB):
      for c in range(HG):
        sl = pl.ds(c * L, L)
        accs = [jnp.maximum(rows_v[i * MAX_NB + r, sl], 0.0) for r in range(4)]
        for r in range(4, MAX_NB):
          accs[r % 4] = accs[r % 4] + jnp.maximum(rows_v[i * MAX_NB + r, sl], 0.0)
        out_v[i, sl] = (accs[0] + accs[1]) + (accs[2] + accs[3])
    pltpu.sync_copy(out_v, agg_hbm.at[pl.ds(a0, SUB)])


_agg_call = pl.kernel(
    _agg_body,
    out_type=jax.ShapeDtypeStruct((A_PAD, HIDDEN), jnp.float32),
    mesh=_sc_mesh(),
    scratch_types=[
        pltpu.VMEM((AGG_ROWS,), jnp.int32),
        pltpu.VMEM((AGG_ROWS, HIDDEN), jnp.float32),
        pltpu.VMEM((SUB, HIDDEN), jnp.float32),
        pltpu.SemaphoreType.DMA,
    ],
)


def _bond_body(z_hbm, agg_hbm, b2a_hbm, b2revb_hbm, m_hbm,
               ia_v, ir_v, agg_v, rev_v, sem):
  base = _wid() * B_PER_W

  @pl.loop(0, BOND_ITERS)
  def _(j):
    b0 = base + j * CH
    pltpu.sync_copy(b2a_hbm.at[pl.ds(b0, CH)], ia_v)
    pltpu.sync_copy(b2revb_hbm.at[pl.ds(b0, CH)], ir_v)
    pltpu.async_copy(agg_hbm.at[ia_v], agg_v, sem).wait()
    pltpu.async_copy(z_hbm.at[ir_v], rev_v, sem).wait()

    @pl.loop(0, CH, unroll=8)
    def _(i):
      for c in range(HG):
        sl = pl.ds(c * L, L)
        agg_v[i, sl] = agg_v[i, sl] - jnp.maximum(rev_v[i, sl], 0.0)

    pltpu.sync_copy(agg_v, m_hbm.at[pl.ds(b0, CH)])


_bond_call = pl.kernel(
    _bond_body,
    out_type=jax.ShapeDtypeStruct((N_BONDS, HIDDEN), jnp.float32),
    mesh=_sc_mesh(),
    scratch_types=[
        pltpu.VMEM((CH,), jnp.int32),
        pltpu.VMEM((CH,), jnp.int32),
        pltpu.VMEM((CH, HIDDEN), jnp.float32),
        pltpu.VMEM((CH, HIDDEN), jnp.float32),
        pltpu.SemaphoreType.DMA,
    ],
)


RB = 2560
GRID = N_BONDS // RB


def _mm_in_body(fb_ref, wi_ref, bi_ref, z_ref):
  z_ref[...] = (
      jnp.dot(fb_ref[...], wi_ref[...], preferred_element_type=jnp.float32)
      + bi_ref[...])


def _mm_in(fb, wi, bi):
  return pl.pallas_call(
      _mm_in_body,
      grid=(GRID,),
      in_specs=[
          pl.BlockSpec((RB, BOND_FDIM), lambda i: (i, 0)),
          pl.BlockSpec((BOND_FDIM, HIDDEN), lambda i: (0, 0)),
          pl.BlockSpec((1, HIDDEN), lambda i: (0, 0)),
      ],
      out_specs=pl.BlockSpec((RB, HIDDEN), lambda i: (i, 0)),
      out_shape=jax.ShapeDtypeStruct((N_BONDS, HIDDEN), jnp.float32),
  )(fb, wi, bi)


def _mm_upd_body(z0_ref, m_ref, wm_ref, bm_ref, z_ref):
  z_ref[...] = (
      z0_ref[...]
      + jnp.dot(m_ref[...], wm_ref[...], preferred_element_type=jnp.float32)
      + bm_ref[...])


def _mm_upd(z0, m, wm, bm):
  return pl.pallas_call(
      _mm_upd_body,
      grid=(GRID,),
      in_specs=[
          pl.BlockSpec((RB, HIDDEN), lambda i: (i, 0)),
          pl.BlockSpec((RB, HIDDEN), lambda i: (i, 0)),
          pl.BlockSpec((HIDDEN, HIDDEN), lambda i: (0, 0)),
          pl.BlockSpec((1, HIDDEN), lambda i: (0, 0)),
      ],
      out_specs=pl.BlockSpec((RB, HIDDEN), lambda i: (i, 0)),
      out_shape=jax.ShapeDtypeStruct((N_BONDS, HIDDEN), jnp.float32),
  )(z0, m, wm, bm)


def _atom_body(fa_ref, am_ref, wa1_ref, wa2_ref, ba_ref, p_ref, out_ref):
  ah = jax.nn.relu(
      jnp.dot(fa_ref[...], wa1_ref[...], preferred_element_type=jnp.float32)
      + jnp.dot(am_ref[...], wa2_ref[...], preferred_element_type=jnp.float32)
      + ba_ref[...])
  out_ref[...] = jnp.dot(p_ref[...], ah, preferred_element_type=jnp.float32)


def _atom_call(fa, am, wa1, wa2, ba, p):
  return pl.pallas_call(
      _atom_body,
      grid=(1,),
      in_specs=[
          pl.BlockSpec((N_ATOMS, ATOM_FDIM), lambda i: (0, 0)),
          pl.BlockSpec((N_ATOMS, HIDDEN), lambda i: (0, 0)),
          pl.BlockSpec((ATOM_FDIM, HIDDEN), lambda i: (0, 0)),
          pl.BlockSpec((HIDDEN, HIDDEN), lambda i: (0, 0)),
          pl.BlockSpec((1, HIDDEN), lambda i: (0, 0)),
          pl.BlockSpec((N_MOLS, N_ATOMS), lambda i: (0, 0)),
      ],
      out_specs=pl.BlockSpec((N_MOLS, HIDDEN), lambda i: (0, 0)),
      out_shape=jax.ShapeDtypeStruct((N_MOLS, HIDDEN), jnp.float32),
  )(fa, am, wa1, wa2, ba, p)


def kernel(f_atoms, f_bonds, a2b, b2a, b2revb, W_i, b_i, W_m, b_m, W_a, b_a):
  a2b = a2b.astype(jnp.int32)
  b2a = b2a.astype(jnp.int32)
  b2revb = b2revb.astype(jnp.int32)

  a2b_flat = jnp.pad(a2b, ((0, A_PAD - N_ATOMS), (0, 0))).reshape(-1)

  bi = b_i.reshape(1, HIDDEN)
  bm = b_m.reshape(1, HIDDEN)
  ba = b_a.reshape(1, HIDDEN)
  wa1 = W_a[:ATOM_FDIM]
  wa2 = W_a[ATOM_FDIM:]
  pool = jnp.kron(jnp.eye(N_MOLS, dtype=jnp.float32),
                  jnp.ones((1, ATOMS_PER_MOL), jnp.float32)) / ATOMS_PER_MOL

  inp = _mm_in(f_bonds, W_i, bi)
  z = inp
  for _ in range(2):
    agg = _agg_call(z, a2b_flat)
    m = _bond_call(z, agg, b2a, b2revb)
    z = _mm_upd(inp, m, W_m, bm)
  amsg = _agg_call(z, a2b_flat)[:N_ATOMS]
  return _atom_call(f_atoms, amsg, wa1, wa2, ba, pool)


# speedup vs baseline: 1.4583x; 1.4583x over previous
import functools
import jax
import jax.numpy as jnp
from jax import lax
from jax.experimental import pallas as pl
from jax.experimental.pallas import tpu as pltpu
from jax.experimental.pallas import tpu_sc as plsc

N_ATOMS = 10000
N_BONDS = 320000
MAX_NB = 32
ATOM_FDIM = 128
BOND_FDIM = 144
HIDDEN = 128
N_MOLS = 100
ATOMS_PER_MOL = 100

NC, NS, L = 2, 16, 16
NW = NC * NS

A_PAD = 10240
A_PER_W = A_PAD // NW
SUB = 4
AGG_ROWS = SUB * MAX_NB
AGG_ITERS = A_PER_W // SUB

B_PER_W = N_BONDS // NW
CH = 80
BOND_ITERS = B_PER_W // CH

HG = HIDDEN // L


def _sc_mesh():
  return plsc.VectorSubcoreMesh(
      core_axis_name="c", subcore_axis_name="s", num_cores=NC, num_subcores=NS)


def _wid():
  return lax.axis_index("s") * NC + lax.axis_index("c")


def _agg_body(z_hbm, a2b3_hbm, agg_hbm, idx_all, rows0, rows1, out_all,
              gsem0, gsem1):
  base = _wid() * A_PER_W
  pltpu.sync_copy(a2b3_hbm.at[_wid()], idx_all)

  def issue(j, rows, gsem):
    pltpu.async_copy(z_hbm.at[idx_all.at[j]], rows, gsem)

  def wait(j, rows, gsem):
    pltpu.make_async_copy(z_hbm.at[idx_all.at[j]], rows, gsem).wait()

  def reduce(j, rows):
    a0 = j * SUB
    for i in range(SUB):
      for c in range(HG):
        sl = pl.ds(c * L, L)
        accs = [jnp.maximum(rows[i * MAX_NB + r, sl], 0.0) for r in range(4)]
        for r in range(4, MAX_NB):
          accs[r % 4] = accs[r % 4] + jnp.maximum(rows[i * MAX_NB + r, sl], 0.0)
        out_all[a0 + i, sl] = (accs[0] + accs[1]) + (accs[2] + accs[3])

  issue(0, rows0, gsem0)

  @pl.loop(0, AGG_ITERS // 2)
  def _(t):
    j0 = 2 * t
    issue(j0 + 1, rows1, gsem1)
    wait(j0, rows0, gsem0)
    reduce(j0, rows0)

    @pl.when(j0 + 2 < AGG_ITERS)
    def _():
      issue(j0 + 2, rows0, gsem0)

    wait(j0 + 1, rows1, gsem1)
    reduce(j0 + 1, rows1)

  pltpu.sync_copy(out_all, agg_hbm.at[pl.ds(base, A_PER_W)])


_agg_call = pl.kernel(
    _agg_body,
    out_type=jax.ShapeDtypeStruct((A_PAD, HIDDEN), jnp.float32),
    mesh=_sc_mesh(),
    scratch_types=[
        pltpu.VMEM((AGG_ITERS, AGG_ROWS), jnp.int32),
        pltpu.VMEM((AGG_ROWS, HIDDEN), jnp.float32),
        pltpu.VMEM((AGG_ROWS, HIDDEN), jnp.float32),
        pltpu.VMEM((A_PER_W, HIDDEN), jnp.float32),
        pltpu.SemaphoreType.DMA,
        pltpu.SemaphoreType.DMA,
    ],
)


def _bond_body(z_hbm, agg_hbm, b2a3_hbm, b2revb3_hbm, m_hbm,
               ia_all, ir_all, aggv, revv, gsems, wsems):
  wid = _wid()
  base = wid * B_PER_W
  pltpu.sync_copy(b2a3_hbm.at[wid], ia_all)
  pltpu.sync_copy(b2revb3_hbm.at[wid], ir_all)

  def issue(j, p):
    pltpu.async_copy(agg_hbm.at[ia_all.at[j]], aggv[p], gsems[p])
    pltpu.async_copy(z_hbm.at[ir_all.at[j]], revv[p], gsems[p])

  def wait_gather(j, p):
    pltpu.make_async_copy(agg_hbm.at[ia_all.at[j]], aggv[p], gsems[p]).wait()
    pltpu.make_async_copy(z_hbm.at[ir_all.at[j]], revv[p], gsems[p]).wait()

  def wait_wb(j, p):
    pltpu.make_async_copy(
        aggv[p], m_hbm.at[pl.ds(base + j * CH, CH)], wsems[p]).wait()

  def stage(j, p, first_round, do_issue=True):
    q = (p + 1) % 3
    if not first_round:
      wait_wb(j - 2, q)
    if do_issue:
      issue(j + 1, q)
    wait_gather(j, p)

    @pl.loop(0, CH, unroll=8)
    def _(i):
      for c in range(HG):
        sl = pl.ds(c * L, L)
        aggv[p][i, sl] = aggv[p][i, sl] - jnp.maximum(revv[p][i, sl], 0.0)

    pltpu.async_copy(aggv[p], m_hbm.at[pl.ds(base + j * CH, CH)], wsems[p])

  issue(0, 0)
  stage(0, 0, True)
  stage(1, 1, True)
  stage(2, 2, False)

  @pl.loop(1, (BOND_ITERS - 3) // 3 + 1)
  def _(t):
    j0 = 3 * t
    stage(j0, 0, False)
    stage(j0 + 1, 1, False)
    stage(j0 + 2, 2, False)

  stage(BOND_ITERS - 2, 0, False)
  stage(BOND_ITERS - 1, 1, False, do_issue=False)
  wait_wb(BOND_ITERS - 2, 0)
  wait_wb(BOND_ITERS - 1, 1)


def _bond_wrapped(z_hbm, agg_hbm, b2a3_hbm, b2revb3_hbm, m_hbm,
                  ia_all, ir_all, a0, a1, a2, r0, r1, r2, g0, g1, g2,
                  w0, w1, w2):
  _bond_body(z_hbm, agg_hbm, b2a3_hbm, b2revb3_hbm, m_hbm, ia_all, ir_all,
             (a0, a1, a2), (r0, r1, r2), (g0, g1, g2), (w0, w1, w2))


_bond_call = pl.kernel(
    _bond_wrapped,
    out_type=jax.ShapeDtypeStruct((N_BONDS, HIDDEN), jnp.float32),
    mesh=_sc_mesh(),
    scratch_types=(
        [pltpu.VMEM((BOND_ITERS, CH), jnp.int32)] * 2
        + [pltpu.VMEM((CH, HIDDEN), jnp.float32)] * 6
        + [pltpu.SemaphoreType.DMA] * 6
    ),
)


RB = 2560
GRID = N_BONDS // RB


def _mm_in_body(fb_ref, wi_ref, bi_ref, z_ref):
  z_ref[...] = (
      jnp.dot(fb_ref[...], wi_ref[...], preferred_element_type=jnp.float32)
      + bi_ref[...])


def _mm_in(fb, wi, bi):
  return pl.pallas_call(
      _mm_in_body,
      grid=(GRID,),
      in_specs=[
          pl.BlockSpec((RB, BOND_FDIM), lambda i: (i, 0)),
          pl.BlockSpec((BOND_FDIM, HIDDEN), lambda i: (0, 0)),
          pl.BlockSpec((1, HIDDEN), lambda i: (0, 0)),
      ],
      out_specs=pl.BlockSpec((RB, HIDDEN), lambda i: (i, 0)),
      out_shape=jax.ShapeDtypeStruct((N_BONDS, HIDDEN), jnp.float32),
  )(fb, wi, bi)


def _mm_upd_body(z0_ref, m_ref, wm_ref, bm_ref, z_ref):
  z_ref[...] = (
      z0_ref[...]
      + jnp.dot(m_ref[...], wm_ref[...], preferred_element_type=jnp.float32)
      + bm_ref[...])


def _mm_upd(z0, m, wm, bm):
  return pl.pallas_call(
      _mm_upd_body,
      grid=(GRID,),
      in_specs=[
          pl.BlockSpec((RB, HIDDEN), lambda i: (i, 0)),
          pl.BlockSpec((RB, HIDDEN), lambda i: (i, 0)),
          pl.BlockSpec((HIDDEN, HIDDEN), lambda i: (0, 0)),
          pl.BlockSpec((1, HIDDEN), lambda i: (0, 0)),
      ],
      out_specs=pl.BlockSpec((RB, HIDDEN), lambda i: (i, 0)),
      out_shape=jax.ShapeDtypeStruct((N_BONDS, HIDDEN), jnp.float32),
  )(z0, m, wm, bm)


def _atom_body(fa_ref, am_ref, wa1_ref, wa2_ref, ba_ref, p_ref, out_ref):
  ah = jax.nn.relu(
      jnp.dot(fa_ref[...], wa1_ref[...], preferred_element_type=jnp.float32)
      + jnp.dot(am_ref[...], wa2_ref[...], preferred_element_type=jnp.float32)
      + ba_ref[...])
  out_ref[...] = jnp.dot(p_ref[...], ah, preferred_element_type=jnp.float32)


def _atom_call(fa, am, wa1, wa2, ba, p):
  return pl.pallas_call(
      _atom_body,
      grid=(1,),
      in_specs=[
          pl.BlockSpec((N_ATOMS, ATOM_FDIM), lambda i: (0, 0)),
          pl.BlockSpec((N_ATOMS, HIDDEN), lambda i: (0, 0)),
          pl.BlockSpec((ATOM_FDIM, HIDDEN), lambda i: (0, 0)),
          pl.BlockSpec((HIDDEN, HIDDEN), lambda i: (0, 0)),
          pl.BlockSpec((1, HIDDEN), lambda i: (0, 0)),
          pl.BlockSpec((N_MOLS, N_ATOMS), lambda i: (0, 0)),
      ],
      out_specs=pl.BlockSpec((N_MOLS, HIDDEN), lambda i: (0, 0)),
      out_shape=jax.ShapeDtypeStruct((N_MOLS, HIDDEN), jnp.float32),
  )(fa, am, wa1, wa2, ba, p)


def kernel(f_atoms, f_bonds, a2b, b2a, b2revb, W_i, b_i, W_m, b_m, W_a, b_a):
  a2b = a2b.astype(jnp.int32)
  b2a = b2a.astype(jnp.int32)
  b2revb = b2revb.astype(jnp.int32)

  a2b3 = jnp.pad(a2b, ((0, A_PAD - N_ATOMS), (0, 0))).reshape(
      NW, AGG_ITERS, AGG_ROWS)
  b2a3 = b2a.reshape(NW, BOND_ITERS, CH)
  b2revb3 = b2revb.reshape(NW, BOND_ITERS, CH)

  bi = b_i.reshape(1, HIDDEN)
  bm = b_m.reshape(1, HIDDEN)
  ba = b_a.reshape(1, HIDDEN)
  wa1 = W_a[:ATOM_FDIM]
  wa2 = W_a[ATOM_FDIM:]
  pool = jnp.kron(jnp.eye(N_MOLS, dtype=jnp.float32),
                  jnp.ones((1, ATOMS_PER_MOL), jnp.float32)) / ATOMS_PER_MOL

  inp = _mm_in(f_bonds, W_i, bi)
  z = inp
  for _ in range(2):
    agg = _agg_call(z, a2b3)
    m = _bond_call(z, agg, b2a3, b2revb3)
    z = _mm_upd(inp, m, W_m, bm)
  amsg = _agg_call(z, a2b3)[:N_ATOMS]
  return _atom_call(f_atoms, amsg, wa1, wa2, ba, pool)


# speedup vs baseline: 1.7454x; 1.1968x over previous
import functools
import jax
import jax.numpy as jnp
from jax import lax
from jax.experimental import pallas as pl
from jax.experimental.pallas import tpu as pltpu
from jax.experimental.pallas import tpu_sc as plsc

N_ATOMS = 10000
N_BONDS = 320000
MAX_NB = 32
ATOM_FDIM = 128
BOND_FDIM = 144
HIDDEN = 128
N_MOLS = 100
ATOMS_PER_MOL = 100

NC, NS, L = 2, 16, 16
NW = NC * NS

A_PAD = 10240
A_PER_W = A_PAD // NW
SUB = 4
AGG_ROWS = SUB * MAX_NB
AGG_ITERS = A_PER_W // SUB

B_PER_W = N_BONDS // NW
CH = 80
BOND_ITERS = B_PER_W // CH

HG = HIDDEN // L


def _sc_mesh():
  return plsc.VectorSubcoreMesh(
      core_axis_name="c", subcore_axis_name="s", num_cores=NC, num_subcores=NS)


def _wid():
  return lax.axis_index("s") * NC + lax.axis_index("c")


def _agg_body(z_hbm, a2b3_hbm, agg_hbm, idx_all, rows0, rows1, out_all,
              gsem0, gsem1):
  base = _wid() * A_PER_W
  pltpu.sync_copy(a2b3_hbm.at[_wid()], idx_all)

  def issue(j, rows, gsem):
    pltpu.async_copy(z_hbm.at[idx_all.at[j]], rows, gsem)

  def wait(j, rows, gsem):
    pltpu.make_async_copy(z_hbm.at[idx_all.at[j]], rows, gsem).wait()

  def reduce(j, rows):
    a0 = j * SUB
    for i in range(SUB):
      for c in range(HG):
        sl = pl.ds(c * L, L)
        accs = [jnp.maximum(rows[i * MAX_NB + r, sl], 0.0) for r in range(4)]
        for r in range(4, MAX_NB):
          accs[r % 4] = accs[r % 4] + jnp.maximum(rows[i * MAX_NB + r, sl], 0.0)
        out_all[a0 + i, sl] = (accs[0] + accs[1]) + (accs[2] + accs[3])

  issue(0, rows0, gsem0)

  @pl.loop(0, AGG_ITERS // 2)
  def _(t):
    j0 = 2 * t
    issue(j0 + 1, rows1, gsem1)
    wait(j0, rows0, gsem0)
    reduce(j0, rows0)

    @pl.when(j0 + 2 < AGG_ITERS)
    def _():
      issue(j0 + 2, rows0, gsem0)

    wait(j0 + 1, rows1, gsem1)
    reduce(j0 + 1, rows1)

  pltpu.sync_copy(out_all, agg_hbm.at[pl.ds(base, A_PER_W)])


_agg_call = pl.kernel(
    _agg_body,
    out_type=jax.ShapeDtypeStruct((A_PAD, HIDDEN), jnp.float32),
    mesh=_sc_mesh(),
    scratch_types=[
        pltpu.VMEM((AGG_ITERS, AGG_ROWS), jnp.int32),
        pltpu.VMEM((AGG_ROWS, HIDDEN), jnp.float32),
        pltpu.VMEM((AGG_ROWS, HIDDEN), jnp.float32),
        pltpu.VMEM((A_PER_W, HIDDEN), jnp.float32),
        pltpu.SemaphoreType.DMA,
        pltpu.SemaphoreType.DMA,
    ],
)


def _bond_body(z_hbm, agg_hbm, b2a3_hbm, b2revb3_hbm, m_hbm,
               ia_all, ir_all, aggv, revv, mv, gsems, wsems):
  wid = _wid()
  base = wid * B_PER_W
  pltpu.sync_copy(b2a3_hbm.at[wid], ia_all)
  pltpu.sync_copy(b2revb3_hbm.at[wid], ir_all)

  def issue(j, p):
    pltpu.async_copy(agg_hbm.at[ia_all.at[j]], aggv[p], gsems[p])
    pltpu.async_copy(z_hbm.at[ir_all.at[j]], revv[p], gsems[p])

  def wait_gather(j, p):
    pltpu.make_async_copy(agg_hbm.at[ia_all.at[j]], aggv[p], gsems[p]).wait()
    pltpu.make_async_copy(z_hbm.at[ir_all.at[j]], revv[p], gsems[p]).wait()

  def wait_wb(j, p):
    pltpu.make_async_copy(
        mv[p], m_hbm.at[pl.ds(base + j * CH, CH)], wsems[p]).wait()

  def stage(j, p, wait_prev_wb, do_issue=True):
    q = (p + 1) % 3
    if do_issue:
      issue(j + 1, q)
    wait_gather(j, p)
    if wait_prev_wb:
      wait_wb(j - 3, p)

    @plsc.parallel_loop(0, CH, unroll=8)
    def _(i):
      for c in range(HG):
        sl = pl.ds(c * L, L)
        mv[p][i, sl] = aggv[p][i, sl] - jnp.maximum(revv[p][i, sl], 0.0)

    pltpu.async_copy(mv[p], m_hbm.at[pl.ds(base + j * CH, CH)], wsems[p])

  issue(0, 0)
  stage(0, 0, False)
  stage(1, 1, False)
  stage(2, 2, False)

  @pl.loop(1, (BOND_ITERS - 3) // 3 + 1)
  def _(t):
    j0 = 3 * t
    stage(j0, 0, True)
    stage(j0 + 1, 1, True)
    stage(j0 + 2, 2, True)

  stage(BOND_ITERS - 2, 0, True)
  stage(BOND_ITERS - 1, 1, True, do_issue=False)
  wait_wb(BOND_ITERS - 3, 2)
  wait_wb(BOND_ITERS - 2, 0)
  wait_wb(BOND_ITERS - 1, 1)


def _bond_wrapped(z_hbm, agg_hbm, b2a3_hbm, b2revb3_hbm, m_hbm,
                  ia_all, ir_all, a0, a1, a2, r0, r1, r2, m0, m1, m2,
                  g0, g1, g2, w0, w1, w2):
  _bond_body(z_hbm, agg_hbm, b2a3_hbm, b2revb3_hbm, m_hbm, ia_all, ir_all,
             (a0, a1, a2), (r0, r1, r2), (m0, m1, m2), (g0, g1, g2),
             (w0, w1, w2))


_bond_call = pl.kernel(
    _bond_wrapped,
    out_type=jax.ShapeDtypeStruct((N_BONDS, HIDDEN), jnp.float32),
    mesh=_sc_mesh(),
    scratch_types=(
        [pltpu.VMEM((BOND_ITERS, CH), jnp.int32)] * 2
        + [pltpu.VMEM((CH, HIDDEN), jnp.float32)] * 9
        + [pltpu.SemaphoreType.DMA] * 6
    ),
)


RB = 2560
GRID = N_BONDS // RB


def _mm_in_body(fb_ref, wi_ref, bi_ref, z_ref):
  z_ref[...] = (
      jnp.dot(fb_ref[...], wi_ref[...], preferred_element_type=jnp.float32)
      + bi_ref[...])


def _mm_in(fb, wi, bi):
  return pl.pallas_call(
      _mm_in_body,
      grid=(GRID,),
      in_specs=[
          pl.BlockSpec((RB, BOND_FDIM), lambda i: (i, 0)),
          pl.BlockSpec((BOND_FDIM, HIDDEN), lambda i: (0, 0)),
          pl.BlockSpec((1, HIDDEN), lambda i: (0, 0)),
      ],
      out_specs=pl.BlockSpec((RB, HIDDEN), lambda i: (i, 0)),
      out_shape=jax.ShapeDtypeStruct((N_BONDS, HIDDEN), jnp.float32),
  )(fb, wi, bi)


def _mm_upd_body(z0_ref, m_ref, wm_ref, bm_ref, z_ref):
  z_ref[...] = (
      z0_ref[...]
      + jnp.dot(m_ref[...], wm_ref[...], preferred_element_type=jnp.float32)
      + bm_ref[...])


def _mm_upd(z0, m, wm, bm):
  return pl.pallas_call(
      _mm_upd_body,
      grid=(GRID,),
      in_specs=[
          pl.BlockSpec((RB, HIDDEN), lambda i: (i, 0)),
          pl.BlockSpec((RB, HIDDEN), lambda i: (i, 0)),
          pl.BlockSpec((HIDDEN, HIDDEN), lambda i: (0, 0)),
          pl.BlockSpec((1, HIDDEN), lambda i: (0, 0)),
      ],
      out_specs=pl.BlockSpec((RB, HIDDEN), lambda i: (i, 0)),
      out_shape=jax.ShapeDtypeStruct((N_BONDS, HIDDEN), jnp.float32),
  )(z0, m, wm, bm)


def _atom_body(fa_ref, am_ref, wa1_ref, wa2_ref, ba_ref, p_ref, out_ref):
  ah = jax.nn.relu(
      jnp.dot(fa_ref[...], wa1_ref[...], preferred_element_type=jnp.float32)
      + jnp.dot(am_ref[...], wa2_ref[...], preferred_element_type=jnp.float32)
      + ba_ref[...])
  out_ref[...] = jnp.dot(p_ref[...], ah, preferred_element_type=jnp.float32)


def _atom_call(fa, am, wa1, wa2, ba, p):
  return pl.pallas_call(
      _atom_body,
      grid=(1,),
      in_specs=[
          pl.BlockSpec((N_ATOMS, ATOM_FDIM), lambda i: (0, 0)),
          pl.BlockSpec((N_ATOMS, HIDDEN), lambda i: (0, 0)),
          pl.BlockSpec((ATOM_FDIM, HIDDEN), lambda i: (0, 0)),
          pl.BlockSpec((HIDDEN, HIDDEN), lambda i: (0, 0)),
          pl.BlockSpec((1, HIDDEN), lambda i: (0, 0)),
          pl.BlockSpec((N_MOLS, N_ATOMS), lambda i: (0, 0)),
      ],
      out_specs=pl.BlockSpec((N_MOLS, HIDDEN), lambda i: (0, 0)),
      out_shape=jax.ShapeDtypeStruct((N_MOLS, HIDDEN), jnp.float32),
  )(fa, am, wa1, wa2, ba, p)


def kernel(f_atoms, f_bonds, a2b, b2a, b2revb, W_i, b_i, W_m, b_m, W_a, b_a):
  a2b = a2b.astype(jnp.int32)
  b2a = b2a.astype(jnp.int32)
  b2revb = b2revb.astype(jnp.int32)

  a2b3 = jnp.pad(a2b, ((0, A_PAD - N_ATOMS), (0, 0))).reshape(
      NW, AGG_ITERS, AGG_ROWS)
  b2a3 = b2a.reshape(NW, BOND_ITERS, CH)
  b2revb3 = b2revb.reshape(NW, BOND_ITERS, CH)

  bi = b_i.reshape(1, HIDDEN)
  bm = b_m.reshape(1, HIDDEN)
  ba = b_a.reshape(1, HIDDEN)
  wa1 = W_a[:ATOM_FDIM]
  wa2 = W_a[ATOM_FDIM:]
  pool = jnp.kron(jnp.eye(N_MOLS, dtype=jnp.float32),
                  jnp.ones((1, ATOMS_PER_MOL), jnp.float32)) / ATOMS_PER_MOL

  inp = _mm_in(f_bonds, W_i, bi)
  z = inp
  for _ in range(2):
    agg = _agg_call(z, a2b3)
    m = _bond_call(z, agg, b2a3, b2revb3)
    z = _mm_upd(inp, m, W_m, bm)
  amsg = _agg_call(z, a2b3)[:N_ATOMS]
  return _atom_call(f_atoms, amsg, wa1, wa2, ba, pool)


# speedup vs baseline: 1.8343x; 1.0510x over previous
import functools
import jax
import jax.numpy as jnp
from jax import lax
from jax.experimental import pallas as pl
from jax.experimental.pallas import tpu as pltpu
from jax.experimental.pallas import tpu_sc as plsc

N_ATOMS = 10000
N_BONDS = 320000
MAX_NB = 32
ATOM_FDIM = 128
BOND_FDIM = 144
HIDDEN = 128
N_MOLS = 100
ATOMS_PER_MOL = 100

NC, NS, L = 2, 16, 16
NW = NC * NS

A_PAD = 10240
A_PER_W = A_PAD // NW
SUB = 2
AGG_ROWS = SUB * MAX_NB
AGG_ITERS = A_PER_W // SUB

B_PER_W = N_BONDS // NW
CH = 80
BOND_ITERS = B_PER_W // CH

HG = HIDDEN // L


def _sc_mesh():
  return plsc.VectorSubcoreMesh(
      core_axis_name="c", subcore_axis_name="s", num_cores=NC, num_subcores=NS)


def _wid():
  return lax.axis_index("s") * NC + lax.axis_index("c")


def _agg_body(z_hbm, a2b3_hbm, agg_hbm, idx_all, r0, r1, r2, out_all,
              g0, g1, g2):
  base = _wid() * A_PER_W
  pltpu.sync_copy(a2b3_hbm.at[_wid()], idx_all)
  rows = (r0, r1, r2)
  gsems = (g0, g1, g2)

  def issue(j, p):
    pltpu.async_copy(z_hbm.at[idx_all.at[j]], rows[p], gsems[p])

  def wait(j, p):
    pltpu.make_async_copy(z_hbm.at[idx_all.at[j]], rows[p], gsems[p]).wait()

  def reduce(j, p):
    a0 = j * SUB
    for i in range(SUB):
      for c in range(HG):
        sl = pl.ds(c * L, L)
        accs = [jnp.maximum(rows[p][i * MAX_NB + r, sl], 0.0)
                for r in range(4)]
        for r in range(4, MAX_NB):
          accs[r % 4] = accs[r % 4] + jnp.maximum(
              rows[p][i * MAX_NB + r, sl], 0.0)
        out_all[a0 + i, sl] = (accs[0] + accs[1]) + (accs[2] + accs[3])

  def stage(j, p, do_issue=True):
    if do_issue:
      issue(j + 1, (p + 1) % 3)
    wait(j, p)
    reduce(j, p)

  issue(0, 0)

  @pl.loop(0, (AGG_ITERS - 1) // 3)
  def _(t):
    j0 = 3 * t
    stage(j0, 0)
    stage(j0 + 1, 1)
    stage(j0 + 2, 2)

  stage(AGG_ITERS - 1, (AGG_ITERS - 1) % 3, do_issue=False)

  pltpu.sync_copy(out_all, agg_hbm.at[pl.ds(base, A_PER_W)])


_agg_call = pl.kernel(
    _agg_body,
    out_type=jax.ShapeDtypeStruct((A_PAD, HIDDEN), jnp.float32),
    mesh=_sc_mesh(),
    scratch_types=(
        [pltpu.VMEM((AGG_ITERS, AGG_ROWS), jnp.int32)]
        + [pltpu.VMEM((AGG_ROWS, HIDDEN), jnp.float32)] * 3
        + [pltpu.VMEM((A_PER_W, HIDDEN), jnp.float32)]
        + [pltpu.SemaphoreType.DMA] * 3
    ),
)


def _bond_body(z_hbm, agg_hbm, b2a3_hbm, b2revb3_hbm, m_hbm,
               ia_all, ir_all, aggv, revv, mv, gsems, wsems):
  wid = _wid()
  base = wid * B_PER_W
  pltpu.sync_copy(b2a3_hbm.at[wid], ia_all)
  pltpu.sync_copy(b2revb3_hbm.at[wid], ir_all)

  def issue(j, p):
    pltpu.async_copy(agg_hbm.at[ia_all.at[j]], aggv[p], gsems[p])
    pltpu.async_copy(z_hbm.at[ir_all.at[j]], revv[p], gsems[p])

  def wait_gather(j, p):
    pltpu.make_async_copy(agg_hbm.at[ia_all.at[j]], aggv[p], gsems[p]).wait()
    pltpu.make_async_copy(z_hbm.at[ir_all.at[j]], revv[p], gsems[p]).wait()

  def wait_wb(j, p):
    pltpu.make_async_copy(
        mv[p], m_hbm.at[pl.ds(base + j * CH, CH)], wsems[p]).wait()

  def stage(j, p, wait_prev_wb, do_issue=True):
    q = (p + 1) % 3
    if do_issue:
      issue(j + 1, q)
    wait_gather(j, p)
    if wait_prev_wb:
      wait_wb(j - 3, p)

    @plsc.parallel_loop(0, CH, unroll=8)
    def _(i):
      for c in range(HG):
        sl = pl.ds(c * L, L)
        mv[p][i, sl] = aggv[p][i, sl] - jnp.maximum(revv[p][i, sl], 0.0)

    pltpu.async_copy(mv[p], m_hbm.at[pl.ds(base + j * CH, CH)], wsems[p])

  issue(0, 0)
  stage(0, 0, False)
  stage(1, 1, False)
  stage(2, 2, False)

  @pl.loop(1, (BOND_ITERS - 3) // 3 + 1)
  def _(t):
    j0 = 3 * t
    stage(j0, 0, True)
    stage(j0 + 1, 1, True)
    stage(j0 + 2, 2, True)

  stage(BOND_ITERS - 2, 0, True)
  stage(BOND_ITERS - 1, 1, True, do_issue=False)
  wait_wb(BOND_ITERS - 3, 2)
  wait_wb(BOND_ITERS - 2, 0)
  wait_wb(BOND_ITERS - 1, 1)


def _bond_wrapped(z_hbm, agg_hbm, b2a3_hbm, b2revb3_hbm, m_hbm,
                  ia_all, ir_all, a0, a1, a2, r0, r1, r2, m0, m1, m2,
                  g0, g1, g2, w0, w1, w2):
  _bond_body(z_hbm, agg_hbm, b2a3_hbm, b2revb3_hbm, m_hbm, ia_all, ir_all,
             (a0, a1, a2), (r0, r1, r2), (m0, m1, m2), (g0, g1, g2),
             (w0, w1, w2))


_bond_call = pl.kernel(
    _bond_wrapped,
    out_type=jax.ShapeDtypeStruct((N_BONDS, HIDDEN), jnp.float32),
    mesh=_sc_mesh(),
    scratch_types=(
        [pltpu.VMEM((BOND_ITERS, CH), jnp.int32)] * 2
        + [pltpu.VMEM((CH, HIDDEN), jnp.float32)] * 9
        + [pltpu.SemaphoreType.DMA] * 6
    ),
)


RB = 2560
GRID = N_BONDS // RB


def _mm_in_body(fb_ref, wi_ref, bi_ref, z_ref):
  z_ref[...] = (
      jnp.dot(fb_ref[...], wi_ref[...], preferred_element_type=jnp.float32)
      + bi_ref[...])


def _mm_in(fb, wi, bi):
  return pl.pallas_call(
      _mm_in_body,
      grid=(GRID,),
      in_specs=[
          pl.BlockSpec((RB, BOND_FDIM), lambda i: (i, 0)),
          pl.BlockSpec((BOND_FDIM, HIDDEN), lambda i: (0, 0)),
          pl.BlockSpec((1, HIDDEN), lambda i: (0, 0)),
      ],
      out_specs=pl.BlockSpec((RB, HIDDEN), lambda i: (i, 0)),
      out_shape=jax.ShapeDtypeStruct((N_BONDS, HIDDEN), jnp.float32),
  )(fb, wi, bi)


def _mm_upd_body(z0_ref, m_ref, wm_ref, bm_ref, z_ref):
  z_ref[...] = (
      z0_ref[...]
      + jnp.dot(m_ref[...], wm_ref[...], preferred_element_type=jnp.float32)
      + bm_ref[...])


def _mm_upd(z0, m, wm, bm):
  return pl.pallas_call(
      _mm_upd_body,
      grid=(GRID,),
      in_specs=[
          pl.BlockSpec((RB, HIDDEN), lambda i: (i, 0)),
          pl.BlockSpec((RB, HIDDEN), lambda i: (i, 0)),
          pl.BlockSpec((HIDDEN, HIDDEN), lambda i: (0, 0)),
          pl.BlockSpec((1, HIDDEN), lambda i: (0, 0)),
      ],
      out_specs=pl.BlockSpec((RB, HIDDEN), lambda i: (i, 0)),
      out_shape=jax.ShapeDtypeStruct((N_BONDS, HIDDEN), jnp.float32),
  )(z0, m, wm, bm)


def _atom_body(fa_ref, am_ref, wa1_ref, wa2_ref, ba_ref, p_ref, out_ref):
  ah = jax.nn.relu(
      jnp.dot(fa_ref[...], wa1_ref[...], preferred_element_type=jnp.float32)
      + jnp.dot(am_ref[...], wa2_ref[...], preferred_element_type=jnp.float32)
      + ba_ref[...])
  out_ref[...] = jnp.dot(p_ref[...], ah, preferred_element_type=jnp.float32)


def _atom_call(fa, am, wa1, wa2, ba, p):
  return pl.pallas_call(
      _atom_body,
      grid=(1,),
      in_specs=[
          pl.BlockSpec((N_ATOMS, ATOM_FDIM), lambda i: (0, 0)),
          pl.BlockSpec((N_ATOMS, HIDDEN), lambda i: (0, 0)),
          pl.BlockSpec((ATOM_FDIM, HIDDEN), lambda i: (0, 0)),
          pl.BlockSpec((HIDDEN, HIDDEN), lambda i: (0, 0)),
          pl.BlockSpec((1, HIDDEN), lambda i: (0, 0)),
          pl.BlockSpec((N_MOLS, N_ATOMS), lambda i: (0, 0)),
      ],
      out_specs=pl.BlockSpec((N_MOLS, HIDDEN), lambda i: (0, 0)),
      out_shape=jax.ShapeDtypeStruct((N_MOLS, HIDDEN), jnp.float32),
  )(fa, am, wa1, wa2, ba, p)


def kernel(f_atoms, f_bonds, a2b, b2a, b2revb, W_i, b_i, W_m, b_m, W_a, b_a):
  a2b = a2b.astype(jnp.int32)
  b2a = b2a.astype(jnp.int32)
  b2revb = b2revb.astype(jnp.int32)

  a2b3 = jnp.pad(a2b, ((0, A_PAD - N_ATOMS), (0, 0))).reshape(
      NW, AGG_ITERS, AGG_ROWS)
  b2a3 = b2a.reshape(NW, BOND_ITERS, CH)
  b2revb3 = b2revb.reshape(NW, BOND_ITERS, CH)

  bi = b_i.reshape(1, HIDDEN)
  bm = b_m.reshape(1, HIDDEN)
  ba = b_a.reshape(1, HIDDEN)
  wa1 = W_a[:ATOM_FDIM]
  wa2 = W_a[ATOM_FDIM:]
  pool = jnp.kron(jnp.eye(N_MOLS, dtype=jnp.float32),
                  jnp.ones((1, ATOMS_PER_MOL), jnp.float32)) / ATOMS_PER_MOL

  inp = _mm_in(f_bonds, W_i, bi)
  z = inp
  for _ in range(2):
    agg = _agg_call(z, a2b3)
    m = _bond_call(z, agg, b2a3, b2revb3)
    z = _mm_upd(inp, m, W_m, bm)
  amsg = _agg_call(z, a2b3)[:N_ATOMS]
  return _atom_call(f_atoms, amsg, wa1, wa2, ba, pool)


# speedup vs baseline: 1.8373x; 1.0016x over previous
import functools
import jax
import jax.numpy as jnp
from jax import lax
from jax.experimental import pallas as pl
from jax.experimental.pallas import tpu as pltpu
from jax.experimental.pallas import tpu_sc as plsc

N_ATOMS = 10000
N_BONDS = 320000
MAX_NB = 32
ATOM_FDIM = 128
BOND_FDIM = 144
HIDDEN = 128
N_MOLS = 100
ATOMS_PER_MOL = 100

NC, NS, L = 2, 16, 16
NW = NC * NS

A_PAD = 10240
A_PER_W = A_PAD // NW
SUB = 2
AGG_ROWS = SUB * MAX_NB
AGG_ITERS = A_PER_W // SUB

B_PER_W = N_BONDS // NW
CH = 80
BOND_ITERS = B_PER_W // CH

HG = HIDDEN // L


def _sc_mesh():
  return plsc.VectorSubcoreMesh(
      core_axis_name="c", subcore_axis_name="s", num_cores=NC, num_subcores=NS)


def _wid():
  return lax.axis_index("s") * NC + lax.axis_index("c")


def _agg_body(z_hbm, a2b3_hbm, agg_hbm, idx_all, r0, r1, r2, r3, out_all,
              g0, g1, g2, g3):
  base = _wid() * A_PER_W
  pltpu.sync_copy(a2b3_hbm.at[_wid()], idx_all)
  rows = (r0, r1, r2, r3)
  gsems = (g0, g1, g2, g3)
  HB = AGG_ROWS // 2

  def issue(j, p):
    pltpu.async_copy(z_hbm.at[idx_all.at[j, pl.ds(0, HB)]],
                     rows[p].at[pl.ds(0, HB)], gsems[p])
    pltpu.async_copy(z_hbm.at[idx_all.at[j, pl.ds(HB, HB)]],
                     rows[p].at[pl.ds(HB, HB)], gsems[p])

  def wait(j, p):
    pltpu.make_async_copy(z_hbm.at[idx_all.at[j, pl.ds(0, HB)]],
                          rows[p].at[pl.ds(0, HB)], gsems[p]).wait()
    pltpu.make_async_copy(z_hbm.at[idx_all.at[j, pl.ds(HB, HB)]],
                          rows[p].at[pl.ds(HB, HB)], gsems[p]).wait()

  def reduce(j, p):
    a0 = j * SUB
    for i in range(SUB):
      for c in range(HG):
        sl = pl.ds(c * L, L)
        accs = [jnp.maximum(rows[p][i * MAX_NB + r, sl], 0.0)
                for r in range(4)]
        for r in range(4, MAX_NB):
          accs[r % 4] = accs[r % 4] + jnp.maximum(
              rows[p][i * MAX_NB + r, sl], 0.0)
        out_all[a0 + i, sl] = (accs[0] + accs[1]) + (accs[2] + accs[3])

  def stage(j, p, do_issue=True):
    if do_issue:
      issue(j + 2, (p + 2) % 4)
    wait(j, p)
    reduce(j, p)

  issue(0, 0)
  issue(1, 1)

  @pl.loop(0, AGG_ITERS // 4 - 1)
  def _(t):
    j0 = 4 * t
    stage(j0, 0)
    stage(j0 + 1, 1)
    stage(j0 + 2, 2)
    stage(j0 + 3, 3)

  stage(AGG_ITERS - 4, 0)
  stage(AGG_ITERS - 3, 1)
  stage(AGG_ITERS - 2, 2, do_issue=False)
  stage(AGG_ITERS - 1, 3, do_issue=False)

  pltpu.sync_copy(out_all, agg_hbm.at[pl.ds(base, A_PER_W)])


_agg_call = pl.kernel(
    _agg_body,
    out_type=jax.ShapeDtypeStruct((A_PAD, HIDDEN), jnp.float32),
    mesh=_sc_mesh(),
    scratch_types=(
        [pltpu.VMEM((AGG_ITERS, AGG_ROWS), jnp.int32)]
        + [pltpu.VMEM((AGG_ROWS, HIDDEN), jnp.float32)] * 4
        + [pltpu.VMEM((A_PER_W, HIDDEN), jnp.float32)]
        + [pltpu.SemaphoreType.DMA] * 4
    ),
)


def _bond_body(z_hbm, agg_hbm, b2a3_hbm, b2revb3_hbm, m_hbm,
               ia_all, ir_all, aggv, revv, mv, gsems, wsems):
  wid = _wid()
  base = wid * B_PER_W
  pltpu.sync_copy(b2a3_hbm.at[wid], ia_all)
  pltpu.sync_copy(b2revb3_hbm.at[wid], ir_all)

  def issue(j, p):
    pltpu.async_copy(agg_hbm.at[ia_all.at[j]], aggv[p], gsems[p])
    pltpu.async_copy(z_hbm.at[ir_all.at[j]], revv[p], gsems[p])

  def wait_gather(j, p):
    pltpu.make_async_copy(agg_hbm.at[ia_all.at[j]], aggv[p], gsems[p]).wait()
    pltpu.make_async_copy(z_hbm.at[ir_all.at[j]], revv[p], gsems[p]).wait()

  def wait_wb(j, p):
    pltpu.make_async_copy(
        mv[p], m_hbm.at[pl.ds(base + j * CH, CH)], wsems[p]).wait()

  def stage(j, p, wait_prev_wb, do_issue=True):
    q = (p + 1) % 3
    if do_issue:
      issue(j + 1, q)
    wait_gather(j, p)
    if wait_prev_wb:
      wait_wb(j - 3, p)

    @plsc.parallel_loop(0, CH, unroll=8)
    def _(i):
      for c in range(HG):
        sl = pl.ds(c * L, L)
        mv[p][i, sl] = aggv[p][i, sl] - jnp.maximum(revv[p][i, sl], 0.0)

    pltpu.async_copy(mv[p], m_hbm.at[pl.ds(base + j * CH, CH)], wsems[p])

  issue(0, 0)
  stage(0, 0, False)
  stage(1, 1, False)
  stage(2, 2, False)

  @pl.loop(1, (BOND_ITERS - 3) // 3 + 1)
  def _(t):
    j0 = 3 * t
    stage(j0, 0, True)
    stage(j0 + 1, 1, True)
    stage(j0 + 2, 2, True)

  stage(BOND_ITERS - 2, 0, True)
  stage(BOND_ITERS - 1, 1, True, do_issue=False)
  wait_wb(BOND_ITERS - 3, 2)
  wait_wb(BOND_ITERS - 2, 0)
  wait_wb(BOND_ITERS - 1, 1)


def _bond_wrapped(z_hbm, agg_hbm, b2a3_hbm, b2revb3_hbm, m_hbm,
                  ia_all, ir_all, a0, a1, a2, r0, r1, r2, m0, m1, m2,
                  g0, g1, g2, w0, w1, w2):
  _bond_body(z_hbm, agg_hbm, b2a3_hbm, b2revb3_hbm, m_hbm, ia_all, ir_all,
             (a0, a1, a2), (r0, r1, r2), (m0, m1, m2), (g0, g1, g2),
             (w0, w1, w2))


_bond_call = pl.kernel(
    _bond_wrapped,
    out_type=jax.ShapeDtypeStruct((N_BONDS, HIDDEN), jnp.float32),
    mesh=_sc_mesh(),
    scratch_types=(
        [pltpu.VMEM((BOND_ITERS, CH), jnp.int32)] * 2
        + [pltpu.VMEM((CH, HIDDEN), jnp.float32)] * 9
        + [pltpu.SemaphoreType.DMA] * 6
    ),
)


RB = 2560
GRID = N_BONDS // RB


def _mm_in_body(fb_ref, wi_ref, bi_ref, z_ref):
  z_ref[...] = (
      jnp.dot(fb_ref[...], wi_ref[...], preferred_element_type=jnp.float32)
      + bi_ref[...])


def _mm_in(fb, wi, bi):
  return pl.pallas_call(
      _mm_in_body,
      grid=(GRID,),
      in_specs=[
          pl.BlockSpec((RB, BOND_FDIM), lambda i: (i, 0)),
          pl.BlockSpec((BOND_FDIM, HIDDEN), lambda i: (0, 0)),
          pl.BlockSpec((1, HIDDEN), lambda i: (0, 0)),
      ],
      out_specs=pl.BlockSpec((RB, HIDDEN), lambda i: (i, 0)),
      out_shape=jax.ShapeDtypeStruct((N_BONDS, HIDDEN), jnp.float32),
  )(fb, wi, bi)


def _mm_upd_body(z0_ref, m_ref, wm_ref, bm_ref, z_ref):
  z_ref[...] = (
      z0_ref[...]
      + jnp.dot(m_ref[...], wm_ref[...], preferred_element_type=jnp.float32)
      + bm_ref[...])


def _mm_upd(z0, m, wm, bm):
  return pl.pallas_call(
      _mm_upd_body,
      grid=(GRID,),
      in_specs=[
          pl.BlockSpec((RB, HIDDEN), lambda i: (i, 0)),
          pl.BlockSpec((RB, HIDDEN), lambda i: (i, 0)),
          pl.BlockSpec((HIDDEN, HIDDEN), lambda i: (0, 0)),
          pl.BlockSpec((1, HIDDEN), lambda i: (0, 0)),
      ],
      out_specs=pl.BlockSpec((RB, HIDDEN), lambda i: (i, 0)),
      out_shape=jax.ShapeDtypeStruct((N_BONDS, HIDDEN), jnp.float32),
  )(z0, m, wm, bm)


def _atom_body(fa_ref, am_ref, wa1_ref, wa2_ref, ba_ref, p_ref, out_ref):
  ah = jax.nn.relu(
      jnp.dot(fa_ref[...], wa1_ref[...], preferred_element_type=jnp.float32)
      + jnp.dot(am_ref[...], wa2_ref[...], preferred_element_type=jnp.float32)
      + ba_ref[...])
  out_ref[...] = jnp.dot(p_ref[...], ah, preferred_element_type=jnp.float32)


def _atom_call(fa, am, wa1, wa2, ba, p):
  return pl.pallas_call(
      _atom_body,
      grid=(1,),
      in_specs=[
          pl.BlockSpec((N_ATOMS, ATOM_FDIM), lambda i: (0, 0)),
          pl.BlockSpec((N_ATOMS, HIDDEN), lambda i: (0, 0)),
          pl.BlockSpec((ATOM_FDIM, HIDDEN), lambda i: (0, 0)),
          pl.BlockSpec((HIDDEN, HIDDEN), lambda i: (0, 0)),
          pl.BlockSpec((1, HIDDEN), lambda i: (0, 0)),
          pl.BlockSpec((N_MOLS, N_ATOMS), lambda i: (0, 0)),
      ],
      out_specs=pl.BlockSpec((N_MOLS, HIDDEN), lambda i: (0, 0)),
      out_shape=jax.ShapeDtypeStruct((N_MOLS, HIDDEN), jnp.float32),
  )(fa, am, wa1, wa2, ba, p)


def kernel(f_atoms, f_bonds, a2b, b2a, b2revb, W_i, b_i, W_m, b_m, W_a, b_a):
  a2b = a2b.astype(jnp.int32)
  b2a = b2a.astype(jnp.int32)
  b2revb = b2revb.astype(jnp.int32)

  a2b3 = jnp.pad(a2b, ((0, A_PAD - N_ATOMS), (0, 0))).reshape(
      NW, AGG_ITERS, AGG_ROWS)
  b2a3 = b2a.reshape(NW, BOND_ITERS, CH)
  b2revb3 = b2revb.reshape(NW, BOND_ITERS, CH)

  bi = b_i.reshape(1, HIDDEN)
  bm = b_m.reshape(1, HIDDEN)
  ba = b_a.reshape(1, HIDDEN)
  wa1 = W_a[:ATOM_FDIM]
  wa2 = W_a[ATOM_FDIM:]
  pool = jnp.kron(jnp.eye(N_MOLS, dtype=jnp.float32),
                  jnp.ones((1, ATOMS_PER_MOL), jnp.float32)) / ATOMS_PER_MOL

  inp = _mm_in(f_bonds, W_i, bi)
  z = inp
  for _ in range(2):
    agg = _agg_call(z, a2b3)
    m = _bond_call(z, agg, b2a3, b2revb3)
    z = _mm_upd(inp, m, W_m, bm)
  amsg = _agg_call(z, a2b3)[:N_ATOMS]
  return _atom_call(f_atoms, amsg, wa1, wa2, ba, pool)
